# Initial kernel scaffold; baseline (speedup 1.0000x reference)
import jax
import jax.numpy as jnp
from jax.experimental import pallas as pl


def kernel(cell_features, bridge_cell_features, inter_neighborhood, Ws, bs, Wt, bt, Wb, bb, attn_p, Wm, bm, Wo, bo):
    raise NotImplementedError("write your pallas kernel here")


# speedup vs baseline: 7.9757x; 7.9757x over previous
import jax
import jax.numpy as jnp
from jax.experimental import pallas as pl
from jax.experimental.pallas import tpu as pltpu

N, M, D = 512, 16, 128
H, HID = 4, 128
HEAD = D // H
TJ = 64
TI = 64
NJ = N // TJ
NI = N // TI
F32 = jnp.float32


def _prologue(cell_ref, bridge_ref, ws_ref, bs_ref, wt_ref, bt_ref, wb_ref,
              bb_ref, wm_ref, bm_ref, hs_ref, ht_ref, cs_ref, bwb_ref, bwm_ref):
    cell = cell_ref[...]
    bridge = bridge_ref[...]
    hs_ref[...] = jnp.dot(cell, ws_ref[...], preferred_element_type=F32) + bs_ref[...]
    ht_ref[...] = (jnp.dot(cell, wt_ref[...], preferred_element_type=F32)
                   + bt_ref[...] + bb_ref[...])
    cs_ref[...] = jnp.dot(cell, wm_ref[0:D, :], preferred_element_type=F32) + bm_ref[...]
    bwb_ref[...] = jnp.dot(bridge, wb_ref[...], preferred_element_type=F32)
    bwm_ref[...] = jnp.dot(bridge, wm_ref[D:2 * D, :], preferred_element_type=F32)


def _attend(hs_ref, ht_ref, a_ref, cs_ref, bwb_ref, bwm_ref, ap_ref, wo_ref,
            bo_ref, out_ref, lg_ref, re_ref, mk_ref):
    jt = pl.program_id(0)
    aj = a_ref[pl.ds(jt * TJ, TJ), :]
    htb = ht_ref[...]
    ap = ap_ref[...]
    bwb = bwb_ref[...]

    for c in range(NI):
        ai = a_ref[c * TI:(c + 1) * TI, :]
        hs_c = hs_ref[c * TI:(c + 1) * TI, :]
        cnt = jax.lax.dot_general(ai, aj, (((1,), (1,)), ((), ())),
                                  preferred_element_type=F32)
        exists = cnt > 0.5
        rinv = 1.0 / jnp.maximum(cnt, 1.0)
        re_ref[pl.ds(c * TI, TI), :] = jnp.where(exists, rinv, 0.0)
        gi = c * TI + jax.lax.broadcasted_iota(jnp.int32, (TI, TJ), 0)
        gj = jt * TJ + jax.lax.broadcasted_iota(jnp.int32, (TI, TJ), 1)
        mk_ref[pl.ds(c * TI, TI), :] = jnp.where(
            jnp.logical_or(exists, gi == gj), 1.0, 0.0)
        mb = (ai[:, None, :] * aj[None, :, :]).reshape(TI * TJ, M)
        raw = jnp.dot(mb, bwb, preferred_element_type=F32).reshape(TI, TJ, H * HID)
        pre = raw * rinv[:, :, None] + hs_c[:, None, :] + htb[None, :, :]
        hid = 0.5 * pre * (1.0 + jax.lax.erf(pre * 0.7071067811865476))
        for h in range(H):
            lg_ref[h, pl.ds(c * TI, TI), :] = (
                hid[:, :, h * HID:(h + 1) * HID] * ap[h][None, None, :]
            ).sum(axis=2)

    lg = lg_ref[...]
    mask = mk_ref[...] > 0.5
    ree = re_ref[...]
    lgm = jnp.where(mask[None], lg, -1e30)
    mx = lgm.max(axis=1, keepdims=True)
    ex = jnp.exp(lgm - mx)
    den = ex.sum(axis=1)
    a_full = a_ref[...]
    outs = []
    for h in range(H):
        e_h = ex[h]
        g1 = jax.lax.dot_general(e_h, cs_ref[:, h * HEAD:(h + 1) * HEAD],
                                 (((0,), (0,)), ((), ())),
                                 preferred_element_type=F32)
        w_h = e_h * ree
        s_h = jax.lax.dot_general(w_h, a_full, (((0,), (0,)), ((), ())),
                                  preferred_element_type=F32)
        g2 = jnp.dot(aj * s_h, bwm_ref[:, h * HEAD:(h + 1) * HEAD],
                     preferred_element_type=F32)
        outs.append((g1 + g2) / den[h][:, None])
    grouped = jnp.concatenate(outs, axis=1)
    out_ref[...] = (jnp.dot(grouped, wo_ref[...], preferred_element_type=F32)
                    + bo_ref[...])


def kernel(cell_features, bridge_cell_features, inter_neighborhood, Ws, bs,
           Wt, bt, Wb, bb, attn_p, Wm, bm, Wo, bo):
    a = inter_neighborhood.astype(F32)
    bs2, bt2, bb2, bm2, bo2 = (x.reshape(1, -1) for x in (bs, bt, bb, bm, bo))
    hs, ht, cs, bwb, bwm = pl.pallas_call(
        _prologue,
        out_shape=[
            jax.ShapeDtypeStruct((N, H * HID), F32),
            jax.ShapeDtypeStruct((N, H * HID), F32),
            jax.ShapeDtypeStruct((N, D), F32),
            jax.ShapeDtypeStruct((M, H * HID), F32),
            jax.ShapeDtypeStruct((M, D), F32),
        ],
    )(cell_features, bridge_cell_features, Ws, bs2, Wt, bt2, Wb, bb2, Wm, bm2)
    out = pl.pallas_call(
        _attend,
        grid=(NJ,),
        in_specs=[
            pl.BlockSpec((N, H * HID), lambda j: (0, 0)),
            pl.BlockSpec((TJ, H * HID), lambda j: (j, 0)),
            pl.BlockSpec((N, M), lambda j: (0, 0)),
            pl.BlockSpec((N, D), lambda j: (0, 0)),
            pl.BlockSpec((M, H * HID), lambda j: (0, 0)),
            pl.BlockSpec((M, D), lambda j: (0, 0)),
            pl.BlockSpec((H, HID), lambda j: (0, 0)),
            pl.BlockSpec((D, D), lambda j: (0, 0)),
            pl.BlockSpec((1, D), lambda j: (0, 0)),
        ],
        out_specs=pl.BlockSpec((TJ, D), lambda j: (j, 0)),
        out_shape=jax.ShapeDtypeStruct((N, D), F32),
        scratch_shapes=[
            pltpu.VMEM((H, N, TJ), F32),
            pltpu.VMEM((N, TJ), F32),
            pltpu.VMEM((N, TJ), F32),
        ],
    )(hs, ht, a, cs, bwb, bwm, attn_p, Wo, bo2)
    return out


# speedup vs baseline: 8.5936x; 1.0775x over previous
import jax
import jax.numpy as jnp
from jax.experimental import pallas as pl
from jax.experimental.pallas import tpu as pltpu

N, M, D = 512, 16, 128
H, HID = 4, 128
HEAD = D // H
TJ = 64
TI = 32
NJ = N // TJ
NI = N // TI
F32 = jnp.float32


def _prologue(cell_ref, bridge_ref, ws_ref, bs_ref, wt_ref, bt_ref, wb_ref,
              bb_ref, wm_ref, bm_ref, hs_ref, ht_ref, cs_ref, bwb_ref, bwm_ref):
    cell = cell_ref[...]
    bridge = bridge_ref[...]
    hs_ref[...] = jnp.dot(cell, ws_ref[...], preferred_element_type=F32) + bs_ref[...]
    ht_ref[...] = (jnp.dot(cell, wt_ref[...], preferred_element_type=F32)
                   + bt_ref[...] + bb_ref[...])
    cs_ref[...] = jnp.dot(cell, wm_ref[0:D, :], preferred_element_type=F32) + bm_ref[...]
    bwb_ref[...] = jnp.dot(bridge, wb_ref[...], preferred_element_type=F32)
    bwm_ref[...] = jnp.dot(bridge, wm_ref[D:2 * D, :], preferred_element_type=F32)


def _attend(hs_ref, ht_ref, a_ref, cs_ref, bwb_ref, bwm_ref, ap_ref, wo_ref,
            bo_ref, out_ref, lg_ref, re_ref, mk_ref):
    jt = pl.program_id(0)
    aj = a_ref[pl.ds(jt * TJ, TJ), :]
    htb = ht_ref[...]
    ap = ap_ref[...]
    bwb = bwb_ref[...]

    oh_i = (jax.lax.broadcasted_iota(jnp.int32, (TI, TJ, TI), 0)
            == jax.lax.broadcasted_iota(jnp.int32, (TI, TJ, TI), 2)).astype(F32)
    oh_j = (jax.lax.broadcasted_iota(jnp.int32, (TI, TJ, TJ), 1)
            == jax.lax.broadcasted_iota(jnp.int32, (TI, TJ, TJ), 2)).astype(F32)
    oh = jnp.concatenate([oh_i, oh_j], axis=2)

    for c in range(NI):
        ai = a_ref[c * TI:(c + 1) * TI, :]
        hs_c = hs_ref[c * TI:(c + 1) * TI, :]
        cnt = jax.lax.dot_general(ai, aj, (((1,), (1,)), ((), ())),
                                  preferred_element_type=F32)
        exists = cnt > 0.5
        rinv = 1.0 / jnp.maximum(cnt, 1.0)
        re_ref[pl.ds(c * TI, TI), :] = jnp.where(exists, rinv, 0.0)
        gi = c * TI + jax.lax.broadcasted_iota(jnp.int32, (TI, TJ), 0)
        gj = jt * TJ + jax.lax.broadcasted_iota(jnp.int32, (TI, TJ), 1)
        mk_ref[pl.ds(c * TI, TI), :] = jnp.where(
            jnp.logical_or(exists, gi == gj), 1.0, 0.0)
        mbm = ai[:, None, :] * aj[None, :, :] * rinv[:, :, None]
        feat = jnp.concatenate([mbm, oh], axis=2).reshape(TI * TJ, M + TI + TJ)
        b2 = jnp.concatenate([bwb, hs_c, htb], axis=0)
        pre = jnp.dot(feat, b2, preferred_element_type=F32).reshape(TI, TJ, H * HID)
        hid = 0.5 * pre * (1.0 + jax.lax.erf(pre * 0.7071067811865476))
        for h in range(H):
            lg_ref[h, pl.ds(c * TI, TI), :] = (
                hid[:, :, h * HID:(h + 1) * HID] * ap[h][None, None, :]
            ).sum(axis=2)

    lg = lg_ref[...]
    mask = mk_ref[...] > 0.5
    ree = re_ref[...]
    lgm = jnp.where(mask[None], lg, -1e30)
    mx = lgm.max(axis=1, keepdims=True)
    ex = jnp.exp(lgm - mx)
    den = ex.sum(axis=1)
    a_full = a_ref[...]
    outs = []
    for h in range(H):
        e_h = ex[h]
        g1 = jax.lax.dot_general(e_h, cs_ref[:, h * HEAD:(h + 1) * HEAD],
                                 (((0,), (0,)), ((), ())),
                                 preferred_element_type=F32)
        w_h = e_h * ree
        s_h = jax.lax.dot_general(w_h, a_full, (((0,), (0,)), ((), ())),
                                  preferred_element_type=F32)
        g2 = jnp.dot(aj * s_h, bwm_ref[:, h * HEAD:(h + 1) * HEAD],
                     preferred_element_type=F32)
        outs.append((g1 + g2) / den[h][:, None])
    grouped = jnp.concatenate(outs, axis=1)
    out_ref[...] = (jnp.dot(grouped, wo_ref[...], preferred_element_type=F32)
                    + bo_ref[...])


def kernel(cell_features, bridge_cell_features, inter_neighborhood, Ws, bs,
           Wt, bt, Wb, bb, attn_p, Wm, bm, Wo, bo):
    a = inter_neighborhood.astype(F32)
    bs2, bt2, bb2, bm2, bo2 = (x.reshape(1, -1) for x in (bs, bt, bb, bm, bo))
    hs, ht, cs, bwb, bwm = pl.pallas_call(
        _prologue,
        out_shape=[
            jax.ShapeDtypeStruct((N, H * HID), F32),
            jax.ShapeDtypeStruct((N, H * HID), F32),
            jax.ShapeDtypeStruct((N, D), F32),
            jax.ShapeDtypeStruct((M, H * HID), F32),
            jax.ShapeDtypeStruct((M, D), F32),
        ],
    )(cell_features, bridge_cell_features, Ws, bs2, Wt, bt2, Wb, bb2, Wm, bm2)
    out = pl.pallas_call(
        _attend,
        grid=(NJ,),
        in_specs=[
            pl.BlockSpec((N, H * HID), lambda j: (0, 0)),
            pl.BlockSpec((TJ, H * HID), lambda j: (j, 0)),
            pl.BlockSpec((N, M), lambda j: (0, 0)),
            pl.BlockSpec((N, D), lambda j: (0, 0)),
            pl.BlockSpec((M, H * HID), lambda j: (0, 0)),
            pl.BlockSpec((M, D), lambda j: (0, 0)),
            pl.BlockSpec((H, HID), lambda j: (0, 0)),
            pl.BlockSpec((D, D), lambda j: (0, 0)),
            pl.BlockSpec((1, D), lambda j: (0, 0)),
        ],
        out_specs=pl.BlockSpec((TJ, D), lambda j: (j, 0)),
        out_shape=jax.ShapeDtypeStruct((N, D), F32),
        scratch_shapes=[
            pltpu.VMEM((H, N, TJ), F32),
            pltpu.VMEM((N, TJ), F32),
            pltpu.VMEM((N, TJ), F32),
        ],
        compiler_params=pltpu.CompilerParams(
            dimension_semantics=("parallel",)),
    )(hs, ht, a, cs, bwb, bwm, attn_p, Wo, bo2)
    return out


# speedup vs baseline: 8.8205x; 1.0264x over previous
import jax
import jax.numpy as jnp
from jax.experimental import pallas as pl
from jax.experimental.pallas import tpu as pltpu

N, M, D = 512, 16, 128
H, HID = 4, 128
HEAD = D // H
TJ = 64
TI = 32
NJ = N // TJ
NI = N // TI
F32 = jnp.float32


def _prologue(cell_ref, bridge_ref, ws_ref, bs_ref, wt_ref, bt_ref, wb_ref,
              bb_ref, wm_ref, bm_ref, hs_ref, ht_ref, cs_ref, bwb_ref, bwm_ref):
    cell = cell_ref[...]
    bridge = bridge_ref[...]
    C = 0.7071067811865476
    hs_ref[...] = (jnp.dot(cell, ws_ref[...], preferred_element_type=F32)
                   + bs_ref[...]) * C
    ht_ref[...] = (jnp.dot(cell, wt_ref[...], preferred_element_type=F32)
                   + bt_ref[...] + bb_ref[...]) * C
    cs_ref[...] = jnp.dot(cell, wm_ref[0:D, :], preferred_element_type=F32) + bm_ref[...]
    bwb_ref[...] = jnp.dot(bridge, wb_ref[...], preferred_element_type=F32) * C
    bwm_ref[...] = jnp.dot(bridge, wm_ref[D:2 * D, :], preferred_element_type=F32)


def _attend(hs_ref, ht_ref, a_ref, cs_ref, bwb_ref, bwm_ref, ap_ref, wo_ref,
            bo_ref, out_ref, lg_ref, re_ref, mk_ref, f_ref, b2_ref):
    jt = pl.program_id(0)
    aj = a_ref[pl.ds(jt * TJ, TJ), :]
    aph = ap_ref[...] * 0.7071067811865476

    oh_i = (jax.lax.broadcasted_iota(jnp.int32, (TI, TJ, TI), 0)
            == jax.lax.broadcasted_iota(jnp.int32, (TI, TJ, TI), 2))
    oh_j = (jax.lax.broadcasted_iota(jnp.int32, (TI, TJ, TJ), 1)
            == jax.lax.broadcasted_iota(jnp.int32, (TI, TJ, TJ), 2))
    oh = jnp.concatenate([oh_i.astype(jnp.bfloat16), oh_j.astype(jnp.bfloat16)],
                         axis=2).reshape(TI * TJ, TI + TJ)
    f_ref[:, M:] = oh
    b2_ref[0:M, :] = bwb_ref[...].astype(jnp.bfloat16)
    b2_ref[M + TI:, :] = ht_ref[...].astype(jnp.bfloat16)

    for c in range(NI):
        ai = a_ref[c * TI:(c + 1) * TI, :]
        cnt = jax.lax.dot_general(ai, aj, (((1,), (1,)), ((), ())),
                                  preferred_element_type=F32)
        exists = cnt > 0.5
        rinv = 1.0 / jnp.maximum(cnt, 1.0)
        re_ref[pl.ds(c * TI, TI), :] = jnp.where(exists, rinv, 0.0)
        gi = c * TI + jax.lax.broadcasted_iota(jnp.int32, (TI, TJ), 0)
        gj = jt * TJ + jax.lax.broadcasted_iota(jnp.int32, (TI, TJ), 1)
        mk_ref[pl.ds(c * TI, TI), :] = jnp.where(
            jnp.logical_or(exists, gi == gj), 1.0, 0.0)
        mbm = ai[:, None, :] * aj[None, :, :] * rinv[:, :, None]
        f_ref[:, 0:M] = mbm.reshape(TI * TJ, M).astype(jnp.bfloat16)
        b2_ref[M:M + TI, :] = hs_ref[c * TI:(c + 1) * TI, :].astype(jnp.bfloat16)
        y = jnp.dot(f_ref[...], b2_ref[...],
                    preferred_element_type=F32).reshape(TI, TJ, H * HID)
        g = y + y * jax.lax.erf(y)
        for h in range(H):
            lg_ref[h, pl.ds(c * TI, TI), :] = (
                g[:, :, h * HID:(h + 1) * HID] * aph[h][None, None, :]
            ).sum(axis=2)

    lg = lg_ref[...]
    mask = mk_ref[...] > 0.5
    ree = re_ref[...]
    lgm = jnp.where(mask[None], lg, -1e30)
    mx = lgm.max(axis=1, keepdims=True)
    ex = jnp.exp(lgm - mx)
    den = ex.sum(axis=1)
    a_full = a_ref[...]
    outs = []
    for h in range(H):
        e_h = ex[h]
        g1 = jax.lax.dot_general(e_h, cs_ref[:, h * HEAD:(h + 1) * HEAD],
                                 (((0,), (0,)), ((), ())),
                                 preferred_element_type=F32)
        w_h = e_h * ree
        s_h = jax.lax.dot_general(w_h, a_full, (((0,), (0,)), ((), ())),
                                  preferred_element_type=F32)
        g2 = jnp.dot(aj * s_h, bwm_ref[:, h * HEAD:(h + 1) * HEAD],
                     preferred_element_type=F32)
        outs.append((g1 + g2) / den[h][:, None])
    grouped = jnp.concatenate(outs, axis=1)
    out_ref[...] = (jnp.dot(grouped, wo_ref[...], preferred_element_type=F32)
                    + bo_ref[...])


def kernel(cell_features, bridge_cell_features, inter_neighborhood, Ws, bs,
           Wt, bt, Wb, bb, attn_p, Wm, bm, Wo, bo):
    a = inter_neighborhood.astype(F32)
    bs2, bt2, bb2, bm2, bo2 = (x.reshape(1, -1) for x in (bs, bt, bb, bm, bo))
    hs, ht, cs, bwb, bwm = pl.pallas_call(
        _prologue,
        out_shape=[
            jax.ShapeDtypeStruct((N, H * HID), F32),
            jax.ShapeDtypeStruct((N, H * HID), F32),
            jax.ShapeDtypeStruct((N, D), F32),
            jax.ShapeDtypeStruct((M, H * HID), F32),
            jax.ShapeDtypeStruct((M, D), F32),
        ],
    )(cell_features, bridge_cell_features, Ws, bs2, Wt, bt2, Wb, bb2, Wm, bm2)
    out = pl.pallas_call(
        _attend,
        grid=(NJ,),
        in_specs=[
            pl.BlockSpec((N, H * HID), lambda j: (0, 0)),
            pl.BlockSpec((TJ, H * HID), lambda j: (j, 0)),
            pl.BlockSpec((N, M), lambda j: (0, 0)),
            pl.BlockSpec((N, D), lambda j: (0, 0)),
            pl.BlockSpec((M, H * HID), lambda j: (0, 0)),
            pl.BlockSpec((M, D), lambda j: (0, 0)),
            pl.BlockSpec((H, HID), lambda j: (0, 0)),
            pl.BlockSpec((D, D), lambda j: (0, 0)),
            pl.BlockSpec((1, D), lambda j: (0, 0)),
        ],
        out_specs=pl.BlockSpec((TJ, D), lambda j: (j, 0)),
        out_shape=jax.ShapeDtypeStruct((N, D), F32),
        scratch_shapes=[
            pltpu.VMEM((H, N, TJ), F32),
            pltpu.VMEM((N, TJ), F32),
            pltpu.VMEM((N, TJ), F32),
            pltpu.VMEM((TI * TJ, M + TI + TJ), jnp.bfloat16),
            pltpu.VMEM((M + TI + TJ, H * HID), jnp.bfloat16),
        ],
        compiler_params=pltpu.CompilerParams(
            dimension_semantics=("parallel",)),
    )(hs, ht, a, cs, bwb, bwm, attn_p, Wo, bo2)
    return out


# speedup vs baseline: 9.0303x; 1.0238x over previous
import jax
import jax.numpy as jnp
from jax.experimental import pallas as pl
from jax.experimental.pallas import tpu as pltpu

N, M, D = 512, 16, 128
H, HID = 4, 128
HEAD = D // H
TJ = 128
TI = 16
NJ = N // TJ
NI = N // TI
KF = M + TI
F32 = jnp.float32
BF16 = jnp.bfloat16
RS2 = 0.7071067811865476


def _prologue(cell_ref, bridge_ref, a_ref, ws_ref, bs_ref, wt_ref, bt_ref,
              wb_ref, bb_ref, wm_ref, bm_ref,
              hst_ref, htt_ref, cs_ref, bwbt_ref, bwm_ref, at_ref):
    cell = cell_ref[...]
    cell_t = cell.T
    bridge = bridge_ref[...]
    hst_ref[...] = (jnp.dot(ws_ref[...].T, cell_t, preferred_element_type=F32)
                    + bs_ref[...].T) * RS2
    htt_ref[...] = (jnp.dot(wt_ref[...].T, cell_t, preferred_element_type=F32)
                    + bt_ref[...].T + bb_ref[...].T) * RS2
    cs_ref[...] = jnp.dot(cell, wm_ref[0:D, :], preferred_element_type=F32) + bm_ref[...]
    bwbt_ref[...] = jnp.dot(bridge, wb_ref[...],
                            preferred_element_type=F32).T * RS2
    bwm_ref[...] = jnp.dot(bridge, wm_ref[D:2 * D, :], preferred_element_type=F32)
    at_ref[...] = a_ref[...].T


def _attend(hst_ref, htt_ref, a_ref, at_ref, cs_ref, bwbt_ref, bwm_ref,
            ap_ref, wo_ref, bo_ref, out_ref, lg_ref, re_ref, mk_ref,
            f_ref, b2_ref):
    jt = pl.program_id(0)
    aj = a_ref[pl.ds(jt * TJ, TJ), :]
    apc = ap_ref[...] * RS2

    eye_i = (jax.lax.broadcasted_iota(jnp.int32, (TI, TI), 0)
             == jax.lax.broadcasted_iota(jnp.int32, (TI, TI), 1)).astype(BF16)
    f_ref[M:, :] = jnp.broadcast_to(eye_i[:, :, None], (TI, TI, TJ)).reshape(
        TI, TI * TJ)
    ajt = at_ref[:, pl.ds(jt * TJ, TJ)]
    ajt_l = jnp.broadcast_to(ajt[:, None, :], (M, TI, TJ)).reshape(M, TI * TJ)
    htt_l = jnp.broadcast_to(htt_ref[:, pl.ds(jt * TJ, TJ)][:, None, :],
                             (H * HID, TI, TJ)).reshape(H * HID, TI * TJ)
    b2_ref[:, 0:M] = bwbt_ref[...].astype(BF16)

    r_i = jax.lax.broadcasted_iota(jnp.int32, (1, TI * TJ), 1) // TJ
    r_j = jax.lax.broadcasted_iota(jnp.int32, (1, TI * TJ), 1) % TJ

    for c in range(NI):
        ait = at_ref[:, c * TI:(c + 1) * TI]
        ait_l = jnp.broadcast_to(ait[:, :, None], (M, TI, TJ)).reshape(
            M, TI * TJ)
        m_t = ait_l * ajt_l
        cnt = jnp.sum(m_t, axis=0, keepdims=True)
        exists = cnt > 0.5
        rinv = 1.0 / jnp.maximum(cnt, 1.0)
        re_ref[pl.ds(c * TI, TI), :] = jnp.where(exists, rinv, 0.0).reshape(TI, TJ)
        eye_f = (c * TI + r_i) == (jt * TJ + r_j)
        mk_ref[pl.ds(c * TI, TI), :] = jnp.where(
            jnp.logical_or(exists, eye_f), 1.0, 0.0).reshape(TI, TJ)
        f_ref[0:M, :] = (m_t * rinv).astype(BF16)
        b2_ref[:, M:] = hst_ref[:, c * TI:(c + 1) * TI].astype(BF16)
        y = jnp.dot(b2_ref[...], f_ref[...], preferred_element_type=F32) + htt_l
        g = (y + y * jax.lax.erf(y)) * apc
        lgt = g.reshape(H, HID, TI * TJ).sum(axis=1)
        lg3 = lgt.reshape(H, TI, TJ)
        for h in range(H):
            lg_ref[h, pl.ds(c * TI, TI), :] = lg3[h]

    lg = lg_ref[...]
    mask = mk_ref[...] > 0.5
    ree = re_ref[...]
    lgm = jnp.where(mask[None], lg, -1e30)
    mx = lgm.max(axis=1, keepdims=True)
    ex = jnp.exp(lgm - mx)
    den = ex.sum(axis=1)
    a_full = a_ref[...]
    outs = []
    for h in range(H):
        e_h = ex[h]
        g1 = jax.lax.dot_general(e_h, cs_ref[:, h * HEAD:(h + 1) * HEAD],
                                 (((0,), (0,)), ((), ())),
                                 preferred_element_type=F32)
        w_h = e_h * ree
        s_h = jax.lax.dot_general(w_h, a_full, (((0,), (0,)), ((), ())),
                                  preferred_element_type=F32)
        g2 = jnp.dot(aj * s_h, bwm_ref[:, h * HEAD:(h + 1) * HEAD],
                     preferred_element_type=F32)
        outs.append((g1 + g2) / den[h][:, None])
    grouped = jnp.concatenate(outs, axis=1)
    out_ref[...] = (jnp.dot(grouped, wo_ref[...], preferred_element_type=F32)
                    + bo_ref[...])


def kernel(cell_features, bridge_cell_features, inter_neighborhood, Ws, bs,
           Wt, bt, Wb, bb, attn_p, Wm, bm, Wo, bo):
    a = inter_neighborhood.astype(F32)
    bs2, bt2, bb2, bm2, bo2 = (x.reshape(1, -1) for x in (bs, bt, bb, bm, bo))
    hst, htt, cs, bwbt, bwm, at = pl.pallas_call(
        _prologue,
        out_shape=[
            jax.ShapeDtypeStruct((H * HID, N), F32),
            jax.ShapeDtypeStruct((H * HID, N), F32),
            jax.ShapeDtypeStruct((N, D), F32),
            jax.ShapeDtypeStruct((H * HID, M), F32),
            jax.ShapeDtypeStruct((M, D), F32),
            jax.ShapeDtypeStruct((M, N), F32),
        ],
    )(cell_features, bridge_cell_features, a, Ws, bs2, Wt, bt2, Wb, bb2, Wm, bm2)
    out = pl.pallas_call(
        _attend,
        grid=(NJ,),
        in_specs=[
            pl.BlockSpec((H * HID, N), lambda j: (0, 0)),
            pl.BlockSpec((H * HID, N), lambda j: (0, 0)),
            pl.BlockSpec((N, M), lambda j: (0, 0)),
            pl.BlockSpec((M, N), lambda j: (0, 0)),
            pl.BlockSpec((N, D), lambda j: (0, 0)),
            pl.BlockSpec((H * HID, M), lambda j: (0, 0)),
            pl.BlockSpec((M, D), lambda j: (0, 0)),
            pl.BlockSpec((H * HID, 1), lambda j: (0, 0)),
            pl.BlockSpec((D, D), lambda j: (0, 0)),
            pl.BlockSpec((1, D), lambda j: (0, 0)),
        ],
        out_specs=pl.BlockSpec((TJ, D), lambda j: (j, 0)),
        out_shape=jax.ShapeDtypeStruct((N, D), F32),
        scratch_shapes=[
            pltpu.VMEM((H, N, TJ), F32),
            pltpu.VMEM((N, TJ), F32),
            pltpu.VMEM((N, TJ), F32),
            pltpu.VMEM((KF, TI * TJ), BF16),
            pltpu.VMEM((H * HID, KF), BF16),
        ],
        compiler_params=pltpu.CompilerParams(
            dimension_semantics=("parallel",)),
    )(hst, htt, a, at, cs, bwbt, bwm, attn_p.reshape(H * HID, 1), Wo, bo2)
    return out


# speedup vs baseline: 11.0764x; 1.2266x over previous
import jax
import jax.numpy as jnp
from jax.experimental import pallas as pl
from jax.experimental.pallas import tpu as pltpu

N, M, D = 512, 16, 128
H, HID = 4, 128
HEAD = D // H
TJ = 128
TI = 16
NJ = N // TJ
NI = N // TI
KF = M + TI
F32 = jnp.float32
BF16 = jnp.bfloat16
RS2 = 0.7071067811865476


def _prologue(cell_ref, bridge_ref, a_ref, ws_ref, bs_ref, wt_ref, bt_ref,
              wb_ref, bb_ref, wm_ref, bm_ref,
              hst_ref, htt_ref, cs_ref, bwbt_ref, bwm_ref, at_ref):
    cell = cell_ref[...]
    cell_t = cell.T
    bridge = bridge_ref[...]
    hst_ref[...] = (jnp.dot(ws_ref[...].T, cell_t, preferred_element_type=F32)
                    + bs_ref[...].T) * RS2
    htt_ref[...] = (jnp.dot(wt_ref[...].T, cell_t, preferred_element_type=F32)
                    + bt_ref[...].T + bb_ref[...].T) * RS2
    cs_ref[...] = jnp.dot(cell, wm_ref[0:D, :], preferred_element_type=F32) + bm_ref[...]
    bwbt_ref[...] = jnp.dot(bridge, wb_ref[...],
                            preferred_element_type=F32).T * RS2
    bwm_ref[...] = jnp.dot(bridge, wm_ref[D:2 * D, :], preferred_element_type=F32)
    at_ref[...] = a_ref[...].T


def _attend(hst_ref, htt_ref, a_ref, at_ref, cs_ref, bwbt_ref, bwm_ref,
            ap_ref, wo_ref, bo_ref, out_ref, lg_ref, re_ref, mk_ref,
            f_ref, b2_ref):
    jt = pl.program_id(0)
    aj = a_ref[pl.ds(jt * TJ, TJ), :]
    apc = ap_ref[...] * RS2

    eye_i = (jax.lax.broadcasted_iota(jnp.int32, (TI, TI), 0)
             == jax.lax.broadcasted_iota(jnp.int32, (TI, TI), 1)).astype(BF16)
    f_ref[M:, :] = jnp.broadcast_to(eye_i[:, :, None], (TI, TI, TJ)).reshape(
        TI, TI * TJ)
    ajt = at_ref[:, pl.ds(jt * TJ, TJ)]
    ajt_l = jnp.broadcast_to(ajt[:, None, :], (M, TI, TJ)).reshape(M, TI * TJ)
    htt_l = jnp.broadcast_to(htt_ref[:, pl.ds(jt * TJ, TJ)][:, None, :],
                             (H * HID, TI, TJ)).reshape(H * HID, TI * TJ)
    apc_b = apc.astype(BF16)
    b2_ref[:, 0:M] = bwbt_ref[...].astype(BF16)

    r_i = jax.lax.broadcasted_iota(jnp.int32, (1, TI * TJ), 1) // TJ
    r_j = jax.lax.broadcasted_iota(jnp.int32, (1, TI * TJ), 1) % TJ

    for c in range(NI):
        ait = at_ref[:, c * TI:(c + 1) * TI]
        ait_l = jnp.broadcast_to(ait[:, :, None], (M, TI, TJ)).reshape(
            M, TI * TJ)
        m_t = ait_l * ajt_l
        cnt = jnp.sum(m_t, axis=0, keepdims=True)
        exists = cnt > 0.5
        rinv = 1.0 / jnp.maximum(cnt, 1.0)
        re_ref[pl.ds(c * TI, TI), :] = jnp.where(exists, rinv, 0.0).reshape(TI, TJ)
        eye_f = (c * TI + r_i) == (jt * TJ + r_j)
        mk_ref[pl.ds(c * TI, TI), :] = jnp.where(
            jnp.logical_or(exists, eye_f), 1.0, 0.0).reshape(TI, TJ)
        f_ref[0:M, :] = (m_t * rinv).astype(BF16)
        b2_ref[:, M:] = hst_ref[:, c * TI:(c + 1) * TI].astype(BF16)
        y = jnp.dot(b2_ref[...], f_ref[...], preferred_element_type=F32) + htt_l
        yb = y.astype(BF16)
        g = (yb + yb * jax.lax.erf(yb)) * apc_b
        lgt = g.astype(F32).reshape(H, HID, TI * TJ).sum(axis=1)
        lg3 = lgt.reshape(H, TI, TJ)
        for h in range(H):
            lg_ref[h, pl.ds(c * TI, TI), :] = lg3[h]

    lg = lg_ref[...]
    mask = mk_ref[...] > 0.5
    ree = re_ref[...]
    lgm = jnp.where(mask[None], lg, -1e30)
    mx = lgm.max(axis=1, keepdims=True)
    ex = jnp.exp(lgm - mx)
    den = ex.sum(axis=1)
    a_full = a_ref[...]
    outs = []
    for h in range(H):
        e_h = ex[h]
        g1 = jax.lax.dot_general(e_h, cs_ref[:, h * HEAD:(h + 1) * HEAD],
                                 (((0,), (0,)), ((), ())),
                                 preferred_element_type=F32)
        w_h = e_h * ree
        s_h = jax.lax.dot_general(w_h, a_full, (((0,), (0,)), ((), ())),
                                  preferred_element_type=F32)
        g2 = jnp.dot(aj * s_h, bwm_ref[:, h * HEAD:(h + 1) * HEAD],
                     preferred_element_type=F32)
        outs.append((g1 + g2) / den[h][:, None])
    grouped = jnp.concatenate(outs, axis=1)
    out_ref[...] = (jnp.dot(grouped, wo_ref[...], preferred_element_type=F32)
                    + bo_ref[...])


def kernel(cell_features, bridge_cell_features, inter_neighborhood, Ws, bs,
           Wt, bt, Wb, bb, attn_p, Wm, bm, Wo, bo):
    a = inter_neighborhood.astype(F32)
    bs2, bt2, bb2, bm2, bo2 = (x.reshape(1, -1) for x in (bs, bt, bb, bm, bo))
    hst, htt, cs, bwbt, bwm, at = pl.pallas_call(
        _prologue,
        out_shape=[
            jax.ShapeDtypeStruct((H * HID, N), F32),
            jax.ShapeDtypeStruct((H * HID, N), F32),
            jax.ShapeDtypeStruct((N, D), F32),
            jax.ShapeDtypeStruct((H * HID, M), F32),
            jax.ShapeDtypeStruct((M, D), F32),
            jax.ShapeDtypeStruct((M, N), F32),
        ],
    )(cell_features, bridge_cell_features, a, Ws, bs2, Wt, bt2, Wb, bb2, Wm, bm2)
    out = pl.pallas_call(
        _attend,
        grid=(NJ,),
        in_specs=[
            pl.BlockSpec((H * HID, N), lambda j: (0, 0)),
            pl.BlockSpec((H * HID, N), lambda j: (0, 0)),
            pl.BlockSpec((N, M), lambda j: (0, 0)),
            pl.BlockSpec((M, N), lambda j: (0, 0)),
            pl.BlockSpec((N, D), lambda j: (0, 0)),
            pl.BlockSpec((H * HID, M), lambda j: (0, 0)),
            pl.BlockSpec((M, D), lambda j: (0, 0)),
            pl.BlockSpec((H * HID, 1), lambda j: (0, 0)),
            pl.BlockSpec((D, D), lambda j: (0, 0)),
            pl.BlockSpec((1, D), lambda j: (0, 0)),
        ],
        out_specs=pl.BlockSpec((TJ, D), lambda j: (j, 0)),
        out_shape=jax.ShapeDtypeStruct((N, D), F32),
        scratch_shapes=[
            pltpu.VMEM((H, N, TJ), F32),
            pltpu.VMEM((N, TJ), F32),
            pltpu.VMEM((N, TJ), F32),
            pltpu.VMEM((KF, TI * TJ), BF16),
            pltpu.VMEM((H * HID, KF), BF16),
        ],
        compiler_params=pltpu.CompilerParams(
            dimension_semantics=("parallel",)),
    )(hst, htt, a, at, cs, bwbt, bwm, attn_p.reshape(H * HID, 1), Wo, bo2)
    return out


# speedup vs baseline: 13.6034x; 1.2281x over previous
import jax
import jax.numpy as jnp
from jax.experimental import pallas as pl
from jax.experimental.pallas import tpu as pltpu

N, M, D = 512, 16, 128
H, HID = 4, 128
HEAD = D // H
TJ = 128
TI = 16
NJ = N // TJ
NI = N // TI
KF = M + TI
F32 = jnp.float32
BF16 = jnp.bfloat16
RS2 = 0.7071067811865476


def _prologue(cell_ref, bridge_ref, a_ref, ws_ref, bs_ref, wt_ref, bt_ref,
              wb_ref, bb_ref, wm_ref, bm_ref,
              hst_ref, htt_ref, cs_ref, bwbt_ref, bwm_ref, at_ref):
    cell = cell_ref[...]
    cell_t = cell.T
    bridge = bridge_ref[...]
    hst_ref[...] = (jnp.dot(ws_ref[...].T, cell_t, preferred_element_type=F32)
                    + bs_ref[...].T) * RS2
    htt_ref[...] = (jnp.dot(wt_ref[...].T, cell_t, preferred_element_type=F32)
                    + bt_ref[...].T + bb_ref[...].T) * RS2
    cs_ref[...] = jnp.dot(cell, wm_ref[0:D, :], preferred_element_type=F32) + bm_ref[...]
    bwbt_ref[...] = jnp.dot(bridge, wb_ref[...],
                            preferred_element_type=F32).T * RS2
    bwm_ref[...] = jnp.dot(bridge, wm_ref[D:2 * D, :], preferred_element_type=F32)
    at_ref[...] = a_ref[...].T


def _attend(hst_ref, htt_ref, a_ref, at_ref, cs_ref, bwbt_ref, bwm_ref,
            ap_ref, wo_ref, bo_ref, out_ref, lg_ref, re_ref, mk_ref,
            f_ref, b2_ref):
    jt = pl.program_id(0)
    aj = a_ref[pl.ds(jt * TJ, TJ), :]
    apc = ap_ref[...] * RS2

    eye_i = (jax.lax.broadcasted_iota(jnp.int32, (TI, TI), 0)
             == jax.lax.broadcasted_iota(jnp.int32, (TI, TI), 1)).astype(BF16)
    f_ref[M:, :] = jnp.broadcast_to(eye_i[:, :, None], (TI, TI, TJ)).reshape(
        TI, TI * TJ)
    ajt = at_ref[:, pl.ds(jt * TJ, TJ)]
    ajt_l = jnp.broadcast_to(ajt[:, None, :], (M, TI, TJ)).reshape(M, TI * TJ)
    htt_l = jnp.broadcast_to(htt_ref[:, pl.ds(jt * TJ, TJ)][:, None, :],
                             (H * HID, TI, TJ)).reshape(H * HID, TI * TJ)
    apc_b = apc.astype(BF16)
    htt_lb = htt_l.astype(BF16)
    b2_ref[:, 0:M] = bwbt_ref[...].astype(BF16)

    r_i = jax.lax.broadcasted_iota(jnp.int32, (1, TI * TJ), 1) // TJ
    r_j = jax.lax.broadcasted_iota(jnp.int32, (1, TI * TJ), 1) % TJ

    for c in range(NI):
        ait = at_ref[:, c * TI:(c + 1) * TI]
        ait_l = jnp.broadcast_to(ait[:, :, None], (M, TI, TJ)).reshape(
            M, TI * TJ)
        m_t = ait_l * ajt_l
        cnt = jnp.sum(m_t, axis=0, keepdims=True)
        exists = cnt > 0.5
        rinv = 1.0 / jnp.maximum(cnt, 1.0)
        re_ref[pl.ds(c * TI, TI), :] = jnp.where(exists, rinv, 0.0).reshape(TI, TJ)
        eye_f = (c * TI + r_i) == (jt * TJ + r_j)
        mk_ref[pl.ds(c * TI, TI), :] = jnp.where(
            jnp.logical_or(exists, eye_f), 1.0, 0.0).reshape(TI, TJ)
        f_ref[0:M, :] = (m_t * rinv).astype(BF16)
        b2_ref[:, M:] = hst_ref[:, c * TI:(c + 1) * TI].astype(BF16)
        yb = (jnp.dot(b2_ref[...], f_ref[...],
                      preferred_element_type=F32).astype(BF16) + htt_lb)
        g = (yb + yb * jax.lax.erf(yb)) * apc_b
        lgt = g.astype(F32).reshape(H, HID, TI * TJ).sum(axis=1)
        lg3 = lgt.reshape(H, TI, TJ)
        for h in range(H):
            lg_ref[h, pl.ds(c * TI, TI), :] = lg3[h]

    lg = lg_ref[...]
    mask = mk_ref[...] > 0.5
    ree = re_ref[...]
    lgm = jnp.where(mask[None], lg, -1e30)
    mx = lgm.max(axis=1, keepdims=True)
    ex = jnp.exp(lgm - mx)
    den = ex.sum(axis=1)
    a_full = a_ref[...]
    outs = []
    for h in range(H):
        e_h = ex[h]
        g1 = jax.lax.dot_general(e_h, cs_ref[:, h * HEAD:(h + 1) * HEAD],
                                 (((0,), (0,)), ((), ())),
                                 preferred_element_type=F32)
        w_h = e_h * ree
        s_h = jax.lax.dot_general(w_h, a_full, (((0,), (0,)), ((), ())),
                                  preferred_element_type=F32)
        g2 = jnp.dot(aj * s_h, bwm_ref[:, h * HEAD:(h + 1) * HEAD],
                     preferred_element_type=F32)
        outs.append((g1 + g2) / den[h][:, None])
    grouped = jnp.concatenate(outs, axis=1)
    out_ref[...] = (jnp.dot(grouped, wo_ref[...], preferred_element_type=F32)
                    + bo_ref[...])


def kernel(cell_features, bridge_cell_features, inter_neighborhood, Ws, bs,
           Wt, bt, Wb, bb, attn_p, Wm, bm, Wo, bo):
    a = inter_neighborhood.astype(F32)
    bs2, bt2, bb2, bm2, bo2 = (x.reshape(1, -1) for x in (bs, bt, bb, bm, bo))
    hst, htt, cs, bwbt, bwm, at = pl.pallas_call(
        _prologue,
        out_shape=[
            jax.ShapeDtypeStruct((H * HID, N), F32),
            jax.ShapeDtypeStruct((H * HID, N), F32),
            jax.ShapeDtypeStruct((N, D), F32),
            jax.ShapeDtypeStruct((H * HID, M), F32),
            jax.ShapeDtypeStruct((M, D), F32),
            jax.ShapeDtypeStruct((M, N), F32),
        ],
    )(cell_features, bridge_cell_features, a, Ws, bs2, Wt, bt2, Wb, bb2, Wm, bm2)
    out = pl.pallas_call(
        _attend,
        grid=(NJ,),
        in_specs=[
            pl.BlockSpec((H * HID, N), lambda j: (0, 0)),
            pl.BlockSpec((H * HID, N), lambda j: (0, 0)),
            pl.BlockSpec((N, M), lambda j: (0, 0)),
            pl.BlockSpec((M, N), lambda j: (0, 0)),
            pl.BlockSpec((N, D), lambda j: (0, 0)),
            pl.BlockSpec((H * HID, M), lambda j: (0, 0)),
            pl.BlockSpec((M, D), lambda j: (0, 0)),
            pl.BlockSpec((H * HID, 1), lambda j: (0, 0)),
            pl.BlockSpec((D, D), lambda j: (0, 0)),
            pl.BlockSpec((1, D), lambda j: (0, 0)),
        ],
        out_specs=pl.BlockSpec((TJ, D), lambda j: (j, 0)),
        out_shape=jax.ShapeDtypeStruct((N, D), F32),
        scratch_shapes=[
            pltpu.VMEM((H, N, TJ), F32),
            pltpu.VMEM((N, TJ), F32),
            pltpu.VMEM((N, TJ), F32),
            pltpu.VMEM((KF, TI * TJ), BF16),
            pltpu.VMEM((H * HID, KF), BF16),
        ],
        compiler_params=pltpu.CompilerParams(
            dimension_semantics=("parallel",)),
    )(hst, htt, a, at, cs, bwbt, bwm, attn_p.reshape(H * HID, 1), Wo, bo2)
    return out
